# Initial kernel scaffold; baseline (speedup 1.0000x reference)
import jax
import jax.numpy as jnp
from jax.experimental import pallas as pl


def kernel(move_indices, learnable_embeddings, non_learnable_embeddings):
    raise NotImplementedError("write your pallas kernel here")


# speedup vs baseline: 16.0901x; 16.0901x over previous
import functools

import numpy as np
import jax
import jax.numpy as jnp
from jax import lax
from jax.experimental import pallas as pl
from jax.experimental.pallas import tpu as pltpu
from jax.experimental.pallas import tpu_sc as plsc

EMB_DIM = 256
N_LEARNABLE = 215
NUM_MOVES = 1000
R_PAD = 1024

_rng = np.random.default_rng(0)
_perm = _rng.permutation(EMB_DIM)
_learn_idx = np.sort(_perm[:N_LEARNABLE])
_nonlearn_idx = np.sort(_perm[N_LEARNABLE:])
_inv = np.zeros(EMB_DIM, np.int32)
_inv[_learn_idx] = np.arange(N_LEARNABLE, dtype=np.int32)
_inv[_nonlearn_idx] = N_LEARNABLE + np.arange(EMB_DIM - N_LEARNABLE, dtype=np.int32)
_INVPERM = jnp.asarray(_inv)

NC, NS = 2, 16
NW = NC * NS
B = 4096 * 50
B_PER_W = B // NW
CHUNK = 128
N_CHUNKS = B_PER_W // CHUNK
ROWS_PER_SUB = R_PAD // NS


def _body(tab_hbm, inv_hbm, idx_hbm, out_hbm,
          shared, tbuf, mbuf, ivp, idxv, rows, sem):
    c = lax.axis_index("c")
    s = lax.axis_index("s")
    wid = s * NC + c

    row0 = s * ROWS_PER_SUB
    pltpu.sync_copy(inv_hbm, ivp)
    pltpu.sync_copy(tab_hbm.at[pl.ds(row0, ROWS_PER_SUB)], tbuf)

    def row_body(r, _):
        rvec = jnp.full((16,), r, jnp.int32)
        for g in range(EMB_DIM // 16):
            cols = ivp[pl.ds(g * 16, 16)]
            vals = plsc.load_gather(tbuf, [rvec, cols])
            mbuf[r, pl.ds(g * 16, 16)] = vals
        return 0

    lax.fori_loop(0, ROWS_PER_SUB, row_body, 0)
    pltpu.sync_copy(mbuf, shared.at[pl.ds(row0, ROWS_PER_SUB)])
    plsc.subcore_barrier()

    base = wid * B_PER_W

    def chunk_body(j, _):
        pltpu.sync_copy(idx_hbm.at[wid * N_CHUNKS + j], idxv)
        pltpu.async_copy(shared.at[idxv], rows, sem).wait()
        pltpu.sync_copy(rows, out_hbm.at[pl.ds(base + j * CHUNK, CHUNK)])
        return 0

    lax.fori_loop(0, N_CHUNKS, chunk_body, 0)


@jax.jit
def _run(concat_pad, idx2d):
    mesh = plsc.VectorSubcoreMesh(core_axis_name="c", subcore_axis_name="s")
    f = pl.kernel(
        _body,
        out_type=jax.ShapeDtypeStruct((B, EMB_DIM), jnp.float32),
        mesh=mesh,
        scratch_types=[
            pltpu.VMEM_SHARED((R_PAD, EMB_DIM), jnp.float32),
            pltpu.VMEM((ROWS_PER_SUB, EMB_DIM), jnp.float32),
            pltpu.VMEM((ROWS_PER_SUB, EMB_DIM), jnp.float32),
            pltpu.VMEM((EMB_DIM,), jnp.int32),
            pltpu.VMEM((CHUNK,), jnp.int32),
            pltpu.VMEM((CHUNK, EMB_DIM), jnp.float32),
            pltpu.SemaphoreType.DMA,
        ],
        compiler_params=pltpu.CompilerParams(
            use_tc_tiling_on_sc=False, needs_layout_passes=False),
    )
    return f(concat_pad, _INVPERM, idx2d)


def kernel(move_indices, learnable_embeddings, non_learnable_embeddings):
    concat = jnp.concatenate([learnable_embeddings, non_learnable_embeddings], axis=1)
    concat_pad = jnp.pad(concat, ((0, R_PAD - NUM_MOVES), (0, 0)))
    idx2d = move_indices.reshape(B // CHUNK, CHUNK)
    out = _run(concat_pad, idx2d)
    return out.reshape(move_indices.shape + (EMB_DIM,))


# speedup vs baseline: 17.1005x; 1.0628x over previous
import functools

import numpy as np
import jax
import jax.numpy as jnp
from jax import lax
from jax.experimental import pallas as pl
from jax.experimental.pallas import tpu as pltpu
from jax.experimental.pallas import tpu_sc as plsc

EMB_DIM = 256
N_LEARNABLE = 215
NUM_MOVES = 1000
R_PAD = 1024

_rng = np.random.default_rng(0)
_perm = _rng.permutation(EMB_DIM)
_learn_idx = np.sort(_perm[:N_LEARNABLE])
_nonlearn_idx = np.sort(_perm[N_LEARNABLE:])
_inv = np.zeros(EMB_DIM, np.int32)
_inv[_learn_idx] = np.arange(N_LEARNABLE, dtype=np.int32)
_inv[_nonlearn_idx] = N_LEARNABLE + np.arange(EMB_DIM - N_LEARNABLE, dtype=np.int32)
_INVPERM = jnp.asarray(_inv)

NC, NS = 2, 16
NW = NC * NS
B = 4096 * 50
B_PER_W = B // NW
CHUNK = 128
N_CHUNKS = B_PER_W // CHUNK
ROWS_PER_SUB = R_PAD // NS


NBUF = 2


def _body(tab_hbm, inv_hbm, idx_hbm, out_hbm,
          shared, tbuf, mbuf, ivp, idxv,
          rows0, rows1, gsem0, gsem1, wsem0, wsem1):
    c = lax.axis_index("c")
    s = lax.axis_index("s")
    wid = s * NC + c
    rows = (rows0, rows1)
    gsem = (gsem0, gsem1)
    wsem = (wsem0, wsem1)

    row0 = s * ROWS_PER_SUB
    pltpu.sync_copy(inv_hbm, ivp)
    pltpu.sync_copy(tab_hbm.at[pl.ds(row0, ROWS_PER_SUB)], tbuf)

    def row_body(r, _):
        rvec = jnp.full((16,), r, jnp.int32)
        for g in range(EMB_DIM // 16):
            cols = ivp[pl.ds(g * 16, 16)]
            vals = plsc.load_gather(tbuf, [rvec, cols])
            mbuf[r, pl.ds(g * 16, 16)] = vals
        return 0

    lax.fori_loop(0, ROWS_PER_SUB, row_body, 0)
    pltpu.sync_copy(mbuf, shared.at[pl.ds(row0, ROWS_PER_SUB)])

    pltpu.sync_copy(idx_hbm.at[pl.ds(wid * N_CHUNKS, N_CHUNKS)], idxv)
    plsc.subcore_barrier()

    base = wid * B_PER_W

    def g_start(j, b):
        pltpu.async_copy(shared.at[idxv.at[j]], rows[b], gsem[b])

    def g_wait(b):
        pltpu.make_async_copy(shared.at[idxv.at[0]], rows[b], gsem[b]).wait()

    def w_start(j, b):
        pltpu.async_copy(
            rows[b], out_hbm.at[pl.ds(base + j * CHUNK, CHUNK)], wsem[b])

    def w_wait(b):
        pltpu.make_async_copy(
            rows[b], out_hbm.at[pl.ds(base, CHUNK)], wsem[b]).wait()

    for b in range(NBUF):
        g_start(b, b)

    def outer(t, _):
        for b in range(NBUF):
            j = t * NBUF + b
            g_wait(b)
            w_start(j, b)
        for b in range(NBUF):
            j = t * NBUF + b
            w_wait(b)
            g_start(j + NBUF, b)
        return 0

    n_full = N_CHUNKS // NBUF - 1
    lax.fori_loop(0, n_full, outer, 0)
    for b in range(NBUF):
        j = n_full * NBUF + b
        g_wait(b)
        w_start(j, b)
    for b in range(NBUF):
        w_wait(b)


@jax.jit
def _run(concat_pad, idx2d):
    mesh = plsc.VectorSubcoreMesh(core_axis_name="c", subcore_axis_name="s")
    f = pl.kernel(
        _body,
        out_type=jax.ShapeDtypeStruct((B, EMB_DIM), jnp.float32),
        mesh=mesh,
        scratch_types=[
            pltpu.VMEM_SHARED((R_PAD, EMB_DIM), jnp.float32),
            pltpu.VMEM((ROWS_PER_SUB, EMB_DIM), jnp.float32),
            pltpu.VMEM((ROWS_PER_SUB, EMB_DIM), jnp.float32),
            pltpu.VMEM((EMB_DIM,), jnp.int32),
            pltpu.VMEM((N_CHUNKS, CHUNK), jnp.int32),
            pltpu.VMEM((CHUNK, EMB_DIM), jnp.float32),
            pltpu.VMEM((CHUNK, EMB_DIM), jnp.float32),
            pltpu.SemaphoreType.DMA,
            pltpu.SemaphoreType.DMA,
            pltpu.SemaphoreType.DMA,
            pltpu.SemaphoreType.DMA,
        ],
        compiler_params=pltpu.CompilerParams(
            use_tc_tiling_on_sc=False, needs_layout_passes=False),
    )
    return f(concat_pad, _INVPERM, idx2d)


def kernel(move_indices, learnable_embeddings, non_learnable_embeddings):
    concat = jnp.concatenate([learnable_embeddings, non_learnable_embeddings], axis=1)
    concat_pad = jnp.pad(concat, ((0, R_PAD - NUM_MOVES), (0, 0)))
    idx2d = move_indices.reshape(B // CHUNK, CHUNK)
    out = _run(concat_pad, idx2d)
    return out.reshape(move_indices.shape + (EMB_DIM,))


# speedup vs baseline: 54.4770x; 3.1857x over previous
import functools

import numpy as np
import jax
import jax.numpy as jnp
from jax import lax
from jax.experimental import pallas as pl
from jax.experimental.pallas import tpu as pltpu
from jax.experimental.pallas import tpu_sc as plsc

EMB_DIM = 256
N_LEARNABLE = 215
NUM_MOVES = 1000
R_PAD = 1024

_rng = np.random.default_rng(0)
_perm = _rng.permutation(EMB_DIM)
_learn_idx = np.sort(_perm[:N_LEARNABLE])
_nonlearn_idx = np.sort(_perm[N_LEARNABLE:])
_inv = np.zeros(EMB_DIM, np.int32)
_inv[_learn_idx] = np.arange(N_LEARNABLE, dtype=np.int32)
_inv[_nonlearn_idx] = N_LEARNABLE + np.arange(EMB_DIM - N_LEARNABLE, dtype=np.int32)
_INVPERM = jnp.asarray(_inv)

NC, NS = 2, 16
NW = NC * NS
NI, NJ = 4096, 50
B = NI * NJ
B_PER_W = B // NW
CHUNK = 128
N_CHUNKS = B_PER_W // CHUNK
ROWS_PER_SUB = R_PAD // NS
N_PIECES = 2 * R_PAD
TR_PER_CHUNK = CHUNK // 8
NBUF = 2


def _body(tab_hbm, inv_hbm, idx_hbm, out_hbm,
          tabp, tbuf, mbufp, ivp, idx2, idxc,
          bufA0, bufB0, bufA1, bufB1, gsem0, gsem1, wsem0, wsem1):
    c = lax.axis_index("c")
    s = lax.axis_index("s")
    wid = s * NC + c
    bufA = (bufA0, bufA1)
    bufB = (bufB0, bufB1)
    gsem = (gsem0, gsem1)
    wsem = (wsem0, wsem1)

    row0 = s * ROWS_PER_SUB
    pltpu.sync_copy(inv_hbm, ivp)
    half_rows = ROWS_PER_SUB // 2

    def row_body(r, _):
        rvec = jnp.full((16,), r, jnp.int32)
        ploc = (r // 8) * 16 + r % 8
        for g in range(EMB_DIM // 16):
            cols = ivp[pl.ds(g * 16, 16)]
            vals = plsc.load_gather(tbuf, [rvec, cols])
            mbufp[ploc + 8 * (g // 8), pl.ds(16 * (g % 8), 16)] = vals
        return 0

    for half in range(2):
        pltpu.sync_copy(tab_hbm.at[pl.ds(row0 + half * half_rows, half_rows)], tbuf)
        lax.fori_loop(0, half_rows, row_body, 0)
        pltpu.sync_copy(
            mbufp,
            tabp.at[pl.ds(s * 2 * ROWS_PER_SUB + half * 2 * half_rows, 2 * half_rows)])

    pltpu.sync_copy(idx_hbm.at[pl.ds(wid * N_CHUNKS, N_CHUNKS)], idx2)

    lane8 = jnp.arange(16, dtype=jnp.int32) % 8
    hoff8 = (jnp.arange(16, dtype=jnp.int32) // 8) * 8

    def idx_body(j, _):
        jvec = jnp.full((16,), j, jnp.int32)
        for kb in range(2):
            for g16 in range(8):
                g_local = 8 * kb + g16
                r = plsc.load_gather(idx2, [jvec, lane8 + 8 * g_local])
                p = 2 * r - (r & 7) + hoff8
                idxc[2 * j + kb, pl.ds(16 * g16, 16)] = p
        return 0

    lax.fori_loop(0, N_CHUNKS, idx_body, 0)
    plsc.subcore_barrier()

    rbase = wid * N_CHUNKS * 2 * CHUNK

    def g_start(j, b):
        pltpu.async_copy(tabp.at[idxc.at[2 * j]], bufA[b], gsem[b])
        pltpu.async_copy(tabp.at[idxc.at[2 * j + 1]], bufB[b], gsem[b])

    def g_wait(b):
        pltpu.make_async_copy(tabp.at[idxc.at[0]], bufA[b], gsem[b]).wait()
        pltpu.make_async_copy(tabp.at[idxc.at[1]], bufB[b], gsem[b]).wait()

    def w_start(j, b):
        r0 = rbase + j * 2 * CHUNK
        pltpu.async_copy(bufA[b], out_hbm.at[pl.ds(r0, CHUNK)], wsem[b])
        pltpu.async_copy(bufB[b], out_hbm.at[pl.ds(r0 + CHUNK, CHUNK)], wsem[b])

    def w_wait(b):
        pltpu.make_async_copy(bufA[b], out_hbm.at[pl.ds(rbase, CHUNK)], wsem[b]).wait()
        pltpu.make_async_copy(bufB[b], out_hbm.at[pl.ds(rbase, CHUNK)], wsem[b]).wait()

    for b in range(NBUF):
        g_start(b, b)

    def outer(t, _):
        for b in range(NBUF):
            j = t * NBUF + b
            g_wait(b)
            w_start(j, b)
        for b in range(NBUF):
            j = t * NBUF + b
            w_wait(b)
            g_start(j + NBUF, b)
        return 0

    n_full = N_CHUNKS // NBUF - 1
    lax.fori_loop(0, n_full, outer, 0)
    for b in range(NBUF):
        j = n_full * NBUF + b
        g_wait(b)
        w_start(j, b)
    for b in range(NBUF):
        w_wait(b)


@jax.jit
def _run(concat_pad, idx2d):
    mesh = plsc.VectorSubcoreMesh(core_axis_name="c", subcore_axis_name="s")
    f = pl.kernel(
        _body,
        out_type=jax.ShapeDtypeStruct((B * 2, 128), jnp.float32),
        mesh=mesh,
        scratch_types=[
            pltpu.VMEM_SHARED((N_PIECES, 128), jnp.float32),
            pltpu.VMEM((ROWS_PER_SUB // 2, EMB_DIM), jnp.float32),
            pltpu.VMEM((ROWS_PER_SUB, 128), jnp.float32),
            pltpu.VMEM((EMB_DIM,), jnp.int32),
            pltpu.VMEM((N_CHUNKS, CHUNK), jnp.int32),
            pltpu.VMEM((2 * N_CHUNKS, CHUNK), jnp.int32),
            pltpu.VMEM((CHUNK, 128), jnp.float32),
            pltpu.VMEM((CHUNK, 128), jnp.float32),
            pltpu.VMEM((CHUNK, 128), jnp.float32),
            pltpu.VMEM((CHUNK, 128), jnp.float32),
            pltpu.SemaphoreType.DMA,
            pltpu.SemaphoreType.DMA,
            pltpu.SemaphoreType.DMA,
            pltpu.SemaphoreType.DMA,
        ],
        compiler_params=pltpu.CompilerParams(
            use_tc_tiling_on_sc=False, needs_layout_passes=False),
    )
    return f(concat_pad, _INVPERM, idx2d)


def kernel(move_indices, learnable_embeddings, non_learnable_embeddings):
    concat = jnp.concatenate([learnable_embeddings, non_learnable_embeddings], axis=1)
    concat_pad = jnp.pad(concat, ((0, R_PAD - NUM_MOVES), (0, 0)))
    idx2d = move_indices.T.reshape(B // CHUNK, CHUNK)
    out = _run(concat_pad, idx2d)
    y = out.reshape(NJ, NI // 8, 2, 8, 128)
    y = y.transpose(1, 3, 0, 2, 4)
    return y.reshape(NI, NJ, EMB_DIM)


# speedup vs baseline: 57.4298x; 1.0542x over previous
import functools

import numpy as np
import jax
import jax.numpy as jnp
from jax import lax
from jax.experimental import pallas as pl
from jax.experimental.pallas import tpu as pltpu
from jax.experimental.pallas import tpu_sc as plsc

EMB_DIM = 256
N_LEARNABLE = 215
NUM_MOVES = 1000
R_PAD = 1024

_rng = np.random.default_rng(0)
_perm = _rng.permutation(EMB_DIM)
_learn_idx = np.sort(_perm[:N_LEARNABLE])
_nonlearn_idx = np.sort(_perm[N_LEARNABLE:])
_inv = np.zeros(EMB_DIM, np.int32)
_inv[_learn_idx] = np.arange(N_LEARNABLE, dtype=np.int32)
_inv[_nonlearn_idx] = N_LEARNABLE + np.arange(EMB_DIM - N_LEARNABLE, dtype=np.int32)
_INVPERM = jnp.asarray(_inv)

NC, NS = 2, 16
NW = NC * NS
NI, NJ = 4096, 50
B = NI * NJ
B_PER_W = B // NW
CHUNK = 128
N_CHUNKS = B_PER_W // CHUNK
ROWS_PER_SUB = R_PAD // NS
N_PIECES = 2 * R_PAD
TR_PER_CHUNK = CHUNK // 8
NBUF = 2


def _body(tab_hbm, inv_hbm, idx_hbm, out_hbm,
          tabp, tbufa, tbufb, mbufp, ivp, idx2, idxc,
          bufA0, bufB0, bufA1, bufB1,
          gsem0, gsem1, wsem0, wsem1, isem, tsem0, tsem1):
    c = lax.axis_index("c")
    s = lax.axis_index("s")
    wid = s * NC + c
    bufA = (bufA0, bufA1)
    bufB = (bufB0, bufB1)
    gsem = (gsem0, gsem1)
    wsem = (wsem0, wsem1)
    tbuf = (tbufa, tbufb)
    tsem = (tsem0, tsem1)

    row0 = s * ROWS_PER_SUB
    half_rows = ROWS_PER_SUB // 2
    pltpu.async_copy(idx_hbm.at[pl.ds(wid * N_CHUNKS, N_CHUNKS)], idx2, isem)
    for half in range(2):
        pltpu.async_copy(
            tab_hbm.at[pl.ds(row0 + half * half_rows, half_rows)],
            tbuf[half], tsem[half])
    pltpu.sync_copy(inv_hbm, ivp)

    def make_row_body(buf):
        def row_body(r, _):
            rvec = jnp.full((16,), r, jnp.int32)
            ploc = (r // 8) * 16 + r % 8
            for g in range(EMB_DIM // 16):
                cols = ivp[pl.ds(g * 16, 16)]
                vals = plsc.load_gather(buf, [rvec, cols])
                mbufp[ploc + 8 * (g // 8), pl.ds(16 * (g % 8), 16)] = vals
            return 0
        return row_body

    for half in range(2):
        pltpu.make_async_copy(
            tab_hbm.at[pl.ds(row0, half_rows)], tbuf[half], tsem[half]).wait()
        lax.fori_loop(0, half_rows, make_row_body(tbuf[half]), 0)
        pltpu.sync_copy(
            mbufp,
            tabp.at[pl.ds(s * 2 * ROWS_PER_SUB + half * 2 * half_rows, 2 * half_rows)])

    lane8 = jnp.arange(16, dtype=jnp.int32) % 8
    hoff8 = (jnp.arange(16, dtype=jnp.int32) // 8) * 8

    def idx_compute(j):
        jvec = jnp.full((16,), j, jnp.int32)
        for kb in range(2):
            for g16 in range(8):
                g_local = 8 * kb + g16
                r = plsc.load_gather(idx2, [jvec, lane8 + 8 * g_local])
                p = 2 * r - (r & 7) + hoff8
                idxc[2 * j + kb, pl.ds(16 * g16, 16)] = p

    pltpu.make_async_copy(
        idx_hbm.at[pl.ds(0, N_CHUNKS)], idx2, isem).wait()
    for j in range(NBUF):
        idx_compute(j)
    plsc.subcore_barrier()

    rbase = wid * N_CHUNKS * 2 * CHUNK

    def g_start(j, b):
        pltpu.async_copy(tabp.at[idxc.at[2 * j]], bufA[b], gsem[b])
        pltpu.async_copy(tabp.at[idxc.at[2 * j + 1]], bufB[b], gsem[b])

    def g_wait(b):
        pltpu.make_async_copy(tabp.at[idxc.at[0]], bufA[b], gsem[b]).wait()
        pltpu.make_async_copy(tabp.at[idxc.at[1]], bufB[b], gsem[b]).wait()

    def w_start(j, b):
        r0 = rbase + j * 2 * CHUNK
        pltpu.async_copy(bufA[b], out_hbm.at[pl.ds(r0, CHUNK)], wsem[b])
        pltpu.async_copy(bufB[b], out_hbm.at[pl.ds(r0 + CHUNK, CHUNK)], wsem[b])

    def w_wait(b):
        pltpu.make_async_copy(bufA[b], out_hbm.at[pl.ds(rbase, CHUNK)], wsem[b]).wait()
        pltpu.make_async_copy(bufB[b], out_hbm.at[pl.ds(rbase, CHUNK)], wsem[b]).wait()

    for b in range(NBUF):
        g_start(b, b)

    def outer(t, _):
        for b in range(NBUF):
            j = t * NBUF + b
            g_wait(b)
            w_start(j, b)
            idx_compute(jnp.minimum(j + NBUF, N_CHUNKS - 1))
        for b in range(NBUF):
            j = t * NBUF + b
            w_wait(b)
            g_start(j + NBUF, b)
        return 0

    n_full = N_CHUNKS // NBUF - 1
    lax.fori_loop(0, n_full, outer, 0)
    for b in range(NBUF):
        j = n_full * NBUF + b
        g_wait(b)
        w_start(j, b)
    for b in range(NBUF):
        w_wait(b)


@jax.jit
def _run(concat_pad, idx2d):
    mesh = plsc.VectorSubcoreMesh(core_axis_name="c", subcore_axis_name="s")
    f = pl.kernel(
        _body,
        out_type=jax.ShapeDtypeStruct((B * 2, 128), jnp.float32),
        mesh=mesh,
        scratch_types=[
            pltpu.VMEM_SHARED((N_PIECES, 128), jnp.float32),
            pltpu.VMEM((ROWS_PER_SUB // 2, EMB_DIM), jnp.float32),
            pltpu.VMEM((ROWS_PER_SUB // 2, EMB_DIM), jnp.float32),
            pltpu.VMEM((ROWS_PER_SUB, 128), jnp.float32),
            pltpu.VMEM((EMB_DIM,), jnp.int32),
            pltpu.VMEM((N_CHUNKS, CHUNK), jnp.int32),
            pltpu.VMEM((2 * N_CHUNKS, CHUNK), jnp.int32),
            pltpu.VMEM((CHUNK, 128), jnp.float32),
            pltpu.VMEM((CHUNK, 128), jnp.float32),
            pltpu.VMEM((CHUNK, 128), jnp.float32),
            pltpu.VMEM((CHUNK, 128), jnp.float32),
            pltpu.SemaphoreType.DMA,
            pltpu.SemaphoreType.DMA,
            pltpu.SemaphoreType.DMA,
            pltpu.SemaphoreType.DMA,
            pltpu.SemaphoreType.DMA,
            pltpu.SemaphoreType.DMA,
            pltpu.SemaphoreType.DMA,
        ],
        compiler_params=pltpu.CompilerParams(
            use_tc_tiling_on_sc=False, needs_layout_passes=False),
    )
    return f(concat_pad, _INVPERM, idx2d)


def kernel(move_indices, learnable_embeddings, non_learnable_embeddings):
    concat = jnp.concatenate([learnable_embeddings, non_learnable_embeddings], axis=1)
    concat_pad = jnp.pad(concat, ((0, R_PAD - NUM_MOVES), (0, 0)))
    idx2d = move_indices.T.reshape(B // CHUNK, CHUNK)
    out = _run(concat_pad, idx2d)
    y = out.reshape(NJ, NI // 8, 2, 8, 128)
    y = y.transpose(1, 3, 0, 2, 4)
    return y.reshape(NI, NJ, EMB_DIM)
